# Initial kernel scaffold; baseline (speedup 1.0000x reference)
import jax
import jax.numpy as jnp
from jax.experimental import pallas as pl


def kernel(x, edge_index, W1, al1, ar1, b1, W2, al2, ar2, b2):
    raise NotImplementedError("write your pallas kernel here")


# speedup vs baseline: 67.1314x; 67.1314x over previous
import functools

import jax
import jax.numpy as jnp
from jax import lax
from jax.experimental import pallas as pl
from jax.experimental.pallas import tpu as pltpu
from jax.experimental.pallas import tpu_sc as plsc

N_NODES = 10000
N_EDGES = 640000
IN_SIZE = 128
HID = 16
H1 = 8
OUT = 16
NEG = 0.2

NC = 2
NS = 16
NW = NC * NS
EPW = N_EDGES // NW
CHUNK = 80
NCHUNK = EPW // CHUNK
ACC_ROWS = 10240
ROWS_PT = ACC_ROWS // NS
ZROWS = 32

SRC_W = 144
DST_W = 16
ACC_W = 144

SRC2_W = 32
DST2_W = 16
ACC2_W = 32

_DNUMS = lax.GatherDimensionNumbers(
    offset_dims=(), collapsed_slice_dims=(0,), start_index_map=(0,))


def _vgather(v, idx):
    return lax.gather(v, idx[:, None], _DNUMS, slice_sizes=(1,),
                      mode=lax.GatherScatterMode.PROMISE_IN_BOUNDS)


def _bcast_lane(v, lane):
    idx = (lax.iota(jnp.int32, 16) & 0) + lane
    return _vgather(v, idx)


def _hi8(v):
    idx = (lax.iota(jnp.int32, 16) & 7) + 8
    return _vgather(v, idx)


def _stage_a_body(x_ref, w1_ref, alm_ref, arm_ref, tsrc_ref, tdst_ref):
    feat = jnp.dot(x_ref[...], w1_ref[...], preferred_element_type=jnp.float32)
    el = jnp.dot(feat, alm_ref[...], preferred_element_type=jnp.float32)
    er = jnp.dot(feat, arm_ref[...], preferred_element_type=jnp.float32)
    gmax = jnp.max(el, axis=0, keepdims=True)
    z = gmax + er
    c = jnp.maximum(z, NEG * z)
    tsrc_ref[...] = jnp.concatenate([feat, el, jnp.zeros_like(el)], axis=1)
    tdst_ref[...] = jnp.concatenate([er, c], axis=1)


_stage_a = pl.pallas_call(
    _stage_a_body,
    out_shape=[
        jax.ShapeDtypeStruct((N_NODES, SRC_W), jnp.float32),
        jax.ShapeDtypeStruct((N_NODES, DST_W), jnp.float32),
    ],
)


def _edge_pass1_body(tsrc, tdst, src, dst, out,
                     sidx, didx, g1, g2, msg, zbuf, accum, sem1, sem2):
    cid = lax.axis_index("c")
    sid = lax.axis_index("s")
    wid = cid * NS + sid

    def zrow(i, _):
        def zcol(j, _):
            zbuf[i, pl.ds(j * 16, 16)] = jnp.zeros((16,), jnp.float32)
            return 0
        lax.fori_loop(0, ACC_W // 16, zcol, 0)
        return 0
    lax.fori_loop(0, ZROWS, zrow, 0)
    for t in range(ROWS_PT // ZROWS):
        pltpu.sync_copy(zbuf, accum.at[pl.ds(sid * ROWS_PT + t * ZROWS, ZROWS)])
    plsc.subcore_barrier()

    base0 = wid * EPW
    lanelt8 = lax.iota(jnp.int32, 16) < 8

    def chunk_body(j, _):
        base = base0 + j * CHUNK
        pltpu.sync_copy(src.at[pl.ds(base, CHUNK)], sidx)
        pltpu.sync_copy(dst.at[pl.ds(base, CHUNK)], didx)
        cp1 = pltpu.async_copy(tsrc.at[sidx], g1, sem1)
        cp2 = pltpu.async_copy(tdst.at[didx], g2, sem2)
        cp1.wait()
        cp2.wait()

        def edge_body(i, _):
            el = g1[i, pl.ds(IN_SIZE, 16)]
            erc = g2[i, pl.ds(0, 16)]
            s = el + erc
            e = jnp.maximum(s, NEG * s)
            ee = jnp.exp(e - _hi8(erc))
            msg[i, pl.ds(IN_SIZE, 16)] = jnp.where(lanelt8, ee, 0.0)
            for h in range(H1):
                b = _bcast_lane(ee, h)
                msg[i, pl.ds(h * 16, 16)] = g1[i, pl.ds(h * 16, 16)] * b
            return 0
        lax.fori_loop(0, CHUNK, edge_body, 0)
        pltpu.sync_copy(msg, accum.at[didx], add=True)
        return 0
    lax.fori_loop(0, NCHUNK, chunk_body, 0)

    plsc.subcore_barrier()
    pltpu.sync_copy(accum.at[pl.ds(sid * ROWS_PT, ROWS_PT)],
                    out.at[cid, pl.ds(sid * ROWS_PT, ROWS_PT)])


_edge_pass1 = functools.partial(
    pl.kernel,
    out_type=jax.ShapeDtypeStruct((NC, ACC_ROWS, ACC_W), jnp.float32),
    mesh=plsc.VectorSubcoreMesh(core_axis_name="c", subcore_axis_name="s"),
    scratch_types=[
        pltpu.VMEM((CHUNK,), jnp.int32),
        pltpu.VMEM((CHUNK,), jnp.int32),
        pltpu.VMEM((CHUNK, SRC_W), jnp.float32),
        pltpu.VMEM((CHUNK, DST_W), jnp.float32),
        pltpu.VMEM((CHUNK, ACC_W), jnp.float32),
        pltpu.VMEM((ZROWS, ACC_W), jnp.float32),
        pltpu.VMEM_SHARED((ACC_ROWS, ACC_W), jnp.float32),
        pltpu.SemaphoreType.DMA,
        pltpu.SemaphoreType.DMA,
    ],
    compiler_params=pltpu.CompilerParams(use_tc_tiling_on_sc=False),
)(_edge_pass1_body)


def _stage_c_body(p_ref, b1_ref, w2_ref, al2_ref, ar2_ref, e8_ref,
                  p0_ref, p1_ref, tsrc2_ref, tdst2_ref):
    acc = p_ref[0, :N_NODES, :] + p_ref[1, :N_NODES, :]
    msgs = acc[:, :IN_SIZE]
    den = acc[:, IN_SIZE:IN_SIZE + H1]
    deninv = 1.0 / jnp.maximum(den, 1e-16)
    den_exp = jnp.dot(deninv, e8_ref[...], preferred_element_type=jnp.float32)
    rst = msgs * den_exp + b1_ref[...]
    h1 = jnp.where(rst > 0, rst, jnp.exp(rst) - 1.0)
    feat2 = jnp.dot(h1, w2_ref[...], preferred_element_type=jnp.float32)
    el2m = jnp.dot(feat2, al2_ref[...], preferred_element_type=jnp.float32)
    er2m = jnp.dot(feat2, ar2_ref[...], preferred_element_type=jnp.float32)
    gmax = jnp.max(el2m, axis=0, keepdims=True)
    z = gmax + er2m
    c2m = jnp.maximum(z, NEG * z)
    tsrc2_ref[...] = jnp.concatenate([feat2, el2m], axis=1)
    tdst2_ref[...] = (jnp.dot(er2m, p0_ref[...], preferred_element_type=jnp.float32)
                      + jnp.dot(c2m, p1_ref[...], preferred_element_type=jnp.float32))


_stage_c = pl.pallas_call(
    _stage_c_body,
    out_shape=[
        jax.ShapeDtypeStruct((N_NODES, SRC2_W), jnp.float32),
        jax.ShapeDtypeStruct((N_NODES, DST2_W), jnp.float32),
    ],
)


def _edge_pass2_body(tsrc, tdst, src, dst, out,
                     sidx, didx, g1, g2, msg, zbuf, accum, sem1, sem2):
    cid = lax.axis_index("c")
    sid = lax.axis_index("s")
    wid = cid * NS + sid

    def zrow(i, _):
        def zcol(j, _):
            zbuf[i, pl.ds(j * 16, 16)] = jnp.zeros((16,), jnp.float32)
            return 0
        lax.fori_loop(0, ACC2_W // 16, zcol, 0)
        return 0
    lax.fori_loop(0, ZROWS, zrow, 0)
    for t in range(ROWS_PT // ZROWS):
        pltpu.sync_copy(zbuf, accum.at[pl.ds(sid * ROWS_PT + t * ZROWS, ZROWS)])
    plsc.subcore_barrier()

    base0 = wid * EPW
    lane0 = lax.iota(jnp.int32, 16) < 1

    def chunk_body(j, _):
        base = base0 + j * CHUNK
        pltpu.sync_copy(src.at[pl.ds(base, CHUNK)], sidx)
        pltpu.sync_copy(dst.at[pl.ds(base, CHUNK)], didx)
        cp1 = pltpu.async_copy(tsrc.at[sidx], g1, sem1)
        cp2 = pltpu.async_copy(tdst.at[didx], g2, sem2)
        cp1.wait()
        cp2.wait()

        def edge_body(i, _):
            vfe = g1[i, pl.ds(0, 16)]
            vel = g1[i, pl.ds(16, 16)]
            verc = g2[i, pl.ds(0, 16)]
            s = vel + verc
            e = jnp.maximum(s, NEG * s)
            ee = jnp.exp(e - _bcast_lane(verc, 1))
            eb = _bcast_lane(ee, 0)
            msg[i, pl.ds(0, 16)] = vfe * eb
            msg[i, pl.ds(16, 16)] = jnp.where(lane0, eb, 0.0)
            return 0
        lax.fori_loop(0, CHUNK, edge_body, 0)
        pltpu.sync_copy(msg, accum.at[didx], add=True)
        return 0
    lax.fori_loop(0, NCHUNK, chunk_body, 0)

    plsc.subcore_barrier()
    pltpu.sync_copy(accum.at[pl.ds(sid * ROWS_PT, ROWS_PT)],
                    out.at[cid, pl.ds(sid * ROWS_PT, ROWS_PT)])


_edge_pass2 = functools.partial(
    pl.kernel,
    out_type=jax.ShapeDtypeStruct((NC, ACC_ROWS, ACC2_W), jnp.float32),
    mesh=plsc.VectorSubcoreMesh(core_axis_name="c", subcore_axis_name="s"),
    scratch_types=[
        pltpu.VMEM((CHUNK,), jnp.int32),
        pltpu.VMEM((CHUNK,), jnp.int32),
        pltpu.VMEM((CHUNK, SRC2_W), jnp.float32),
        pltpu.VMEM((CHUNK, DST2_W), jnp.float32),
        pltpu.VMEM((CHUNK, ACC2_W), jnp.float32),
        pltpu.VMEM((ZROWS, ACC2_W), jnp.float32),
        pltpu.VMEM_SHARED((ACC_ROWS, ACC2_W), jnp.float32),
        pltpu.SemaphoreType.DMA,
        pltpu.SemaphoreType.DMA,
    ],
    compiler_params=pltpu.CompilerParams(use_tc_tiling_on_sc=False),
)(_edge_pass2_body)


def _stage_e_body(p_ref, b2_ref, bsel_ref, out_ref):
    acc = p_ref[0, :N_NODES, :] + p_ref[1, :N_NODES, :]
    msg2 = acc[:, :OUT]
    den = jnp.dot(acc[:, OUT:], bsel_ref[...], preferred_element_type=jnp.float32)
    out_ref[...] = msg2 / jnp.maximum(den, 1e-16) + b2_ref[...]


_stage_e = pl.pallas_call(
    _stage_e_body,
    out_shape=jax.ShapeDtypeStruct((N_NODES, OUT), jnp.float32),
)


def kernel(x, edge_index, W1, al1, ar1, b1, W2, al2, ar2, b2):
    src = edge_index[0].astype(jnp.int32)
    dst = edge_index[1].astype(jnp.int32)

    eye8 = jnp.eye(H1, dtype=jnp.float32)
    alm1 = (al1[:, :, None] * eye8[:, None, :]).reshape(H1 * HID, H1)
    arm1 = (ar1[:, :, None] * eye8[:, None, :]).reshape(H1 * HID, H1)
    tsrc1, tdst1 = _stage_a(x, W1, alm1, arm1)

    part1 = _edge_pass1(tsrc1, tdst1, src, dst)

    e8 = jnp.kron(eye8, jnp.ones((1, HID), jnp.float32))
    al2m = jnp.zeros((OUT, OUT), jnp.float32).at[:, 0].set(al2[0])
    ar2m = jnp.zeros((OUT, OUT), jnp.float32).at[:, 0].set(ar2[0])
    p0 = jnp.zeros((OUT, DST2_W), jnp.float32).at[0, 0].set(1.0)
    p1 = jnp.zeros((OUT, DST2_W), jnp.float32).at[0, 1].set(1.0)
    tsrc2, tdst2 = _stage_c(part1, b1.reshape(1, H1 * HID), W2, al2m, ar2m,
                            e8, p0, p1)

    part2 = _edge_pass2(tsrc2, tdst2, src, dst)

    bsel = jnp.zeros((OUT, OUT), jnp.float32).at[0, :].set(1.0)
    return _stage_e(part2, b2.reshape(1, OUT), bsel)
